# Initial kernel scaffold; baseline (speedup 1.0000x reference)
import jax
import jax.numpy as jnp
from jax.experimental import pallas as pl


def kernel(edge_index, nfeatures, cars, free, entered, embed, W2p, b2p, W2e, b2e, W3, b3, W4, b4):
    raise NotImplementedError("write your pallas kernel here")


# speedup vs baseline: 144.1269x; 144.1269x over previous
import functools

import jax
import jax.numpy as jnp
from jax import lax
from jax.experimental import pallas as pl
from jax.experimental.pallas import tpu as pltpu
from jax.experimental.pallas import tpu_sc as plsc

N = 100000
E = 1600000
EMB = 32
VOCAB = 1000

N_PAD = 102400
NROWS = N_PAD // 128
NCHN = N_PAD // 512
E_PAD = 1600512
EROWS = E_PAD // 128
NCHE = E_PAD // 1024
SL = N_PAD // 16
PAD_IDX = N_PAD - 1

_f32 = jnp.float32
_i32 = jnp.int32


def _tables_body(embed_ref, w2p_ref, b2p_ref, w2e_ref, b2e_ref, w3_ref,
                 b3_ref, eu_ref, ev_ref, q_ref):
    emb = embed_ref[...]
    w3m = w3_ref[...][:32]
    u = jnp.dot(w2e_ref[...][:32], w3m, preferred_element_type=_f32)
    v = jnp.dot(w2e_ref[...][32:], w3m, preferred_element_type=_f32)
    c8 = jnp.dot(b2e_ref[...].reshape(1, 32), w3m,
                 preferred_element_type=_f32) + b3_ref[...].reshape(1, 8)
    eu_ref[...] = jnp.dot(emb, u, preferred_element_type=_f32) + c8
    ev_ref[...] = jnp.dot(emb, v, preferred_element_type=_f32)
    q_ref[...] = jnp.dot(emb, w2p_ref[...][:32],
                         preferred_element_type=_f32) + b2p_ref[...]


def _make_tables(embed, W2p, b2p, W2e, b2e, W3, b3):
    return pl.pallas_call(
        _tables_body,
        out_shape=(
            jax.ShapeDtypeStruct((VOCAB, 8), _f32),
            jax.ShapeDtypeStruct((VOCAB, 8), _f32),
            jax.ShapeDtypeStruct((VOCAB, 1), _f32),
        ),
    )(embed, W2p, b2p, W2e, b2e, W3, b3)


def _mesh():
    return plsc.VectorSubcoreMesh(core_axis_name="c", subcore_axis_name="s")


def _wid():
    return lax.axis_index("s") * 2 + lax.axis_index("c")


def _zero_shared(zeroblk, sh, base):
    def zb(i, _):
        zeroblk[pl.ds(i * 16, 16)] = jnp.zeros((16,), _f32)
        return 0
    lax.fori_loop(0, 128, zb, 0)
    for off in (0, 2048, 4096):
        pltpu.sync_copy(zeroblk, sh.at[pl.ds(base + off, 2048)])
    pltpu.sync_copy(zeroblk.at[pl.ds(0, 256)], sh.at[pl.ds(base + 6144, 256)])


def _node_body(nf_hbm, cars_hbm, free_hbm, q_hbm, embed_hbm, consts_hbm,
               emb_out, cars1_out, meta_out,
               qtab, cbuf, nfbuf, carsbuf, freebuf, c1buf, mbuf, embrows,
               sem):
    wid = _wid()
    pltpu.sync_copy(q_hbm, qtab)
    pltpu.sync_copy(consts_hbm, cbuf)
    w2p_last = cbuf[5, pl.ds(0, 16)][0]

    def chunk(t, _):
        c = wid + 32 * t

        @pl.when(c < NCHN)
        def _():
            r4 = c * 4
            pltpu.sync_copy(nf_hbm.at[pl.ds(r4, 4)], nfbuf)
            pltpu.sync_copy(cars_hbm.at[pl.ds(r4, 4)], carsbuf)
            pltpu.sync_copy(free_hbm.at[pl.ds(r4, 4)], freebuf)
            descs = [
                pltpu.async_copy(embed_hbm.at[nfbuf.at[j]],
                                 embrows.at[pl.ds(j * 128, 128)], sem)
                for j in range(4)
            ]
            for dsc in descs:
                dsc.wait()
            pltpu.sync_copy(embrows, emb_out.at[pl.ds(c * 512, 512)])

            def grp(g, _):
                j = g // 8
                k = (g % 8) * 16
                nfv = nfbuf[j, pl.ds(k, 16)]
                cv = carsbuf[j, pl.ds(k, 16)]
                fv = freebuf[j, pl.ds(k, 16)]
                qg = plsc.load_gather(qtab, [nfv, jnp.zeros((16,), _i32)])
                parked = qg + cv * w2p_last
                c1 = jnp.maximum(jnp.maximum(parked, 0.0) + cv, 0.0)
                meta = nfv + jnp.where(fv > 0.5, 1024, 0).astype(_i32)
                c1buf[j, pl.ds(k, 16)] = c1
                mbuf[j, pl.ds(k, 16)] = meta
                return 0

            lax.fori_loop(0, 32, grp, 0)
            pltpu.sync_copy(c1buf, cars1_out.at[pl.ds(r4, 4)])
            pltpu.sync_copy(mbuf, meta_out.at[pl.ds(r4, 4)])

        return 0

    lax.fori_loop(0, (NCHN + 31) // 32, chunk, 0)


def _node_stage(nf2d, cars2d, free2d, qtab, embed, consts):
    fn = pl.kernel(
        _node_body,
        out_type=(
            jax.ShapeDtypeStruct((N_PAD, EMB), _f32),
            jax.ShapeDtypeStruct((NROWS, 128), _f32),
            jax.ShapeDtypeStruct((NROWS, 128), _i32),
        ),
        mesh=_mesh(),
        compiler_params=pltpu.CompilerParams(needs_layout_passes=False, use_tc_tiling_on_sc=False),
        scratch_types=(
            pltpu.VMEM((VOCAB, 1), _f32),
            pltpu.VMEM((8, 16), _f32),
            pltpu.VMEM((4, 128), _i32),
            pltpu.VMEM((4, 128), _f32),
            pltpu.VMEM((4, 128), _f32),
            pltpu.VMEM((4, 128), _f32),
            pltpu.VMEM((4, 128), _i32),
            pltpu.VMEM((512, EMB), _f32),
            pltpu.SemaphoreType.DMA,
        ),
    )
    return fn(nf2d, cars2d, free2d, qtab, embed, consts)


def _pass1_body(ep_hbm, c1_hbm, ent_hbm, meta_hbm, eu_hbm, ev_hbm, consts_hbm,
                zs_out, denom_out,
                eutab, evtab, cbuf, sbuf, dbuf, g_c1s, g_ents, g_mets, g_c1d,
                g_metd, zbuf, zpos, zeroblk, gsem, ssem,
                sh_c1, sh_ent, sh_meta, sh_denom):
    cid = lax.axis_index("c")
    sid = lax.axis_index("s")
    wid = sid * 2 + cid
    base = sid * SL

    pltpu.sync_copy(eu_hbm, eutab)
    pltpu.sync_copy(ev_hbm, evtab)
    pltpu.sync_copy(consts_hbm, cbuf)
    pltpu.sync_copy(c1_hbm.at[pl.ds(base, SL)], sh_c1.at[pl.ds(base, SL)])
    pltpu.sync_copy(ent_hbm.at[pl.ds(base, SL)], sh_ent.at[pl.ds(base, SL)])
    pltpu.sync_copy(meta_hbm.at[pl.ds(base, SL)], sh_meta.at[pl.ds(base, SL)])
    _zero_shared(zeroblk, sh_denom, base)
    plsc.subcore_barrier()

    r0 = cbuf[0, pl.ds(0, 16)]
    r1 = cbuf[1, pl.ds(0, 16)]
    r2 = cbuf[2, pl.ds(0, 16)]
    r3 = cbuf[3, pl.ds(0, 16)]
    r4c = cbuf[4, pl.ds(0, 16)]
    w31 = [r0[c] for c in range(8)]
    w32 = [r1[c] for c in range(8)]
    w33 = [r2[c] for c in range(8)]
    w4 = [r3[c] for c in range(8)]
    b4 = r4c[0]

    def chunk(t, _):
        c = wid + 32 * t

        @pl.when(c < NCHE)
        def _():
            row = c * 8
            pltpu.sync_copy(ep_hbm.at[0, pl.ds(row, 8)], sbuf)
            pltpu.sync_copy(ep_hbm.at[1, pl.ds(row, 8)], dbuf)
            descs = []
            for j in range(8):
                descs.append(pltpu.async_copy(
                    sh_c1.at[sbuf.at[j]], g_c1s.at[j], gsem))
                descs.append(pltpu.async_copy(
                    sh_ent.at[sbuf.at[j]], g_ents.at[j], gsem))
                descs.append(pltpu.async_copy(
                    sh_meta.at[sbuf.at[j]], g_mets.at[j], gsem))
                descs.append(pltpu.async_copy(
                    sh_c1.at[dbuf.at[j]], g_c1d.at[j], gsem))
                descs.append(pltpu.async_copy(
                    sh_meta.at[dbuf.at[j]], g_metd.at[j], gsem))
            for dsc in descs:
                dsc.wait()

            def grp(g, _):
                j = g // 8
                k = (g % 8) * 16
                c1s = g_c1s[j, pl.ds(k, 16)]
                ents = g_ents[j, pl.ds(k, 16)]
                mets = g_mets[j, pl.ds(k, 16)]
                c1d = g_c1d[j, pl.ds(k, 16)]
                metd = g_metd[j, pl.ds(k, 16)]
                nfs = jnp.bitwise_and(mets, 1023)
                frees = lax.shift_right_logical(mets, 10)
                nfd = jnp.bitwise_and(metd, 1023)
                h3 = jnp.full((16,), 0.0, _f32) + b4
                for cc in range(8):
                    ccv = jnp.full((16,), cc, _i32)
                    eu = plsc.load_gather(eutab, [nfs, ccv])
                    ev = plsc.load_gather(evtab, [nfd, ccv])
                    a = (eu + ev + c1s * w31[cc] + c1d * w32[cc]
                         + ents * w33[cc])
                    h3 = h3 + jnp.maximum(a, 0.0) * w4[cc]
                lf = jnp.maximum(h3, 0.0)
                slb = nfs == nfd
                enb = slb != (frees == 1)
                lf = jnp.where(enb, lf, 0.0)
                z = jnp.exp(lf)
                zpos[j, pl.ds(k, 16)] = z
                zbuf[j, pl.ds(k, 16)] = jnp.where(slb, -z, z)
                return 0

            lax.fori_loop(0, 64, grp, 0)
            sdescs = [
                pltpu.async_copy(zpos.at[j], sh_denom.at[sbuf.at[j]], ssem,
                                 add=True)
                for j in range(8)
            ]
            for dsc in sdescs:
                dsc.wait()
            pltpu.sync_copy(zbuf, zs_out.at[pl.ds(row, 8)])

        return 0

    lax.fori_loop(0, (NCHE + 31) // 32, chunk, 0)
    plsc.subcore_barrier()
    pltpu.sync_copy(sh_denom.at[pl.ds(base, SL)],
                    denom_out.at[cid, pl.ds(base, SL)])


def _pass1(ep2d, cars1_flat, ent_flat, meta_flat, eutab, evtab, consts):
    fn = pl.kernel(
        _pass1_body,
        out_type=(
            jax.ShapeDtypeStruct((EROWS, 128), _f32),
            jax.ShapeDtypeStruct((2, N_PAD), _f32),
        ),
        mesh=_mesh(),
        compiler_params=pltpu.CompilerParams(needs_layout_passes=False, use_tc_tiling_on_sc=False),
        scratch_types=(
            pltpu.VMEM((VOCAB, 8), _f32),
            pltpu.VMEM((VOCAB, 8), _f32),
            pltpu.VMEM((8, 16), _f32),
            pltpu.VMEM((8, 128), _i32),
            pltpu.VMEM((8, 128), _i32),
            pltpu.VMEM((8, 128), _f32),
            pltpu.VMEM((8, 128), _f32),
            pltpu.VMEM((8, 128), _i32),
            pltpu.VMEM((8, 128), _f32),
            pltpu.VMEM((8, 128), _i32),
            pltpu.VMEM((8, 128), _f32),
            pltpu.VMEM((8, 128), _f32),
            pltpu.VMEM((2048,), _f32),
            pltpu.SemaphoreType.DMA,
            pltpu.SemaphoreType.DMA,
            pltpu.VMEM_SHARED((N_PAD,), _f32),
            pltpu.VMEM_SHARED((N_PAD,), _f32),
            pltpu.VMEM_SHARED((N_PAD,), _i32),
            pltpu.VMEM_SHARED((N_PAD,), _f32),
        ),
    )
    return fn(ep2d, cars1_flat, ent_flat, meta_flat, eutab, evtab, consts)


def _pass2_body(ep_hbm, zs_hbm, c1_hbm, denom_hbm,
                cars_out, ent_out,
                d0buf, d1buf, c1chunk, wbuf, sbuf, dbuf, zsb, wsg, msgb, outb,
                zeroblk, gsem, ssem,
                sh_w, sh_cars, sh_ent):
    cid = lax.axis_index("c")
    sid = lax.axis_index("s")
    wid = sid * 2 + cid
    base = sid * SL

    for off, sz in ((0, 2048), (2048, 2048), (4096, 2048), (6144, 256)):
        pltpu.sync_copy(denom_hbm.at[0, pl.ds(base + off, sz)],
                        d0buf.at[pl.ds(0, sz)])
        pltpu.sync_copy(denom_hbm.at[1, pl.ds(base + off, sz)],
                        d1buf.at[pl.ds(0, sz)])
        pltpu.sync_copy(c1_hbm.at[pl.ds(base + off, sz)],
                        c1chunk.at[pl.ds(0, sz)])

        def wgrp(i, _):
            s16 = pl.ds(i * 16, 16)
            wbuf[s16] = c1chunk[s16] / (d0buf[s16] + d1buf[s16])
            return 0

        lax.fori_loop(0, sz // 16, wgrp, 0)
        pltpu.sync_copy(wbuf.at[pl.ds(0, sz)],
                        sh_w.at[pl.ds(base + off, sz)])

    _zero_shared(zeroblk, sh_cars, base)
    _zero_shared(zeroblk, sh_ent, base)
    plsc.subcore_barrier()

    def chunk(t, _):
        c = wid + 32 * t

        @pl.when(c < NCHE)
        def _():
            row = c * 8
            pltpu.sync_copy(ep_hbm.at[0, pl.ds(row, 8)], sbuf)
            pltpu.sync_copy(ep_hbm.at[1, pl.ds(row, 8)], dbuf)
            pltpu.sync_copy(zs_hbm.at[pl.ds(row, 8)], zsb)
            descs = [
                pltpu.async_copy(sh_w.at[sbuf.at[j]], wsg.at[j], gsem)
                for j in range(8)
            ]
            for dsc in descs:
                dsc.wait()

            def grp(g, _):
                j = g // 8
                k = (g % 8) * 16
                zs16 = zsb[j, pl.ds(k, 16)]
                ws = wsg[j, pl.ds(k, 16)]
                slb = zs16 < 0.0
                msg = jnp.abs(zs16) * ws
                msgb[j, pl.ds(k, 16)] = msg
                outb[j, pl.ds(k, 16)] = jnp.where(slb, 0.0, msg)
                return 0

            lax.fori_loop(0, 64, grp, 0)
            sdescs = []
            for j in range(8):
                sdescs.append(pltpu.async_copy(
                    msgb.at[j], sh_cars.at[dbuf.at[j]], ssem, add=True))
                sdescs.append(pltpu.async_copy(
                    outb.at[j], sh_ent.at[dbuf.at[j]], ssem, add=True))
            for dsc in sdescs:
                dsc.wait()

        return 0

    lax.fori_loop(0, (NCHE + 31) // 32, chunk, 0)
    plsc.subcore_barrier()
    pltpu.sync_copy(sh_cars.at[pl.ds(base, SL)],
                    cars_out.at[cid, pl.ds(base, SL)])
    pltpu.sync_copy(sh_ent.at[pl.ds(base, SL)],
                    ent_out.at[cid, pl.ds(base, SL)])


def _pass2(ep2d, zs2d, cars1_flat, denom2):
    fn = pl.kernel(
        _pass2_body,
        out_type=(
            jax.ShapeDtypeStruct((2, N_PAD), _f32),
            jax.ShapeDtypeStruct((2, N_PAD), _f32),
        ),
        mesh=_mesh(),
        compiler_params=pltpu.CompilerParams(needs_layout_passes=False, use_tc_tiling_on_sc=False),
        scratch_types=(
            pltpu.VMEM((2048,), _f32),
            pltpu.VMEM((2048,), _f32),
            pltpu.VMEM((2048,), _f32),
            pltpu.VMEM((2048,), _f32),
            pltpu.VMEM((8, 128), _i32),
            pltpu.VMEM((8, 128), _i32),
            pltpu.VMEM((8, 128), _f32),
            pltpu.VMEM((8, 128), _f32),
            pltpu.VMEM((8, 128), _f32),
            pltpu.VMEM((8, 128), _f32),
            pltpu.VMEM((2048,), _f32),
            pltpu.SemaphoreType.DMA,
            pltpu.SemaphoreType.DMA,
            pltpu.VMEM_SHARED((N_PAD,), _f32),
            pltpu.VMEM_SHARED((N_PAD,), _f32),
            pltpu.VMEM_SHARED((N_PAD,), _f32),
        ),
    )
    return fn(ep2d, zs2d, cars1_flat, denom2)


def _combine_body(a_ref, b_ref, ca_ref, cb_ref):
    ca_ref[...] = jnp.sum(a_ref[...], axis=0)
    cb_ref[...] = jnp.sum(b_ref[...], axis=0)


def _combine(carsp, entp):
    blk = 2048
    return pl.pallas_call(
        _combine_body,
        grid=(N_PAD // blk,),
        in_specs=[
            pl.BlockSpec((2, blk), lambda i: (0, i)),
            pl.BlockSpec((2, blk), lambda i: (0, i)),
        ],
        out_specs=[
            pl.BlockSpec((blk,), lambda i: (i,)),
            pl.BlockSpec((blk,), lambda i: (i,)),
        ],
        out_shape=(
            jax.ShapeDtypeStruct((N_PAD,), _f32),
            jax.ShapeDtypeStruct((N_PAD,), _f32),
        ),
    )(carsp, entp)


def kernel(edge_index, nfeatures, cars, free, entered, embed,
           W2p, b2p, W2e, b2e, W3, b3, W4, b4):
    eutab, evtab, q2d = _make_tables(embed, W2p, b2p, W2e, b2e, W3, b3)

    consts = jnp.zeros((8, 16), _f32)
    consts = consts.at[0, :8].set(W3[32])
    consts = consts.at[1, :8].set(W3[33])
    consts = consts.at[2, :8].set(W3[34])
    consts = consts.at[3, :8].set(W4[:, 0])
    consts = consts.at[4, 0].set(b4[0])
    consts = consts.at[5, 0].set(W2p[32, 0])

    nf2d = jnp.pad(nfeatures, (0, N_PAD - N)).reshape(NROWS, 128)
    cars2d = jnp.pad(cars[:, 0], (0, N_PAD - N)).reshape(NROWS, 128)
    free2d = jnp.pad(free[:, 0], (0, N_PAD - N)).reshape(NROWS, 128)
    ent_flat = jnp.pad(entered[:, 0], (0, N_PAD - N))

    ep2d = jnp.pad(edge_index, ((0, 0), (0, E_PAD - E)),
                   constant_values=PAD_IDX).reshape(2, EROWS, 128)

    emb_out, cars1_2d, meta2d = _node_stage(nf2d, cars2d, free2d, q2d,
                                            embed, consts)
    cars1_flat = cars1_2d.reshape(N_PAD)
    meta_flat = meta2d.reshape(N_PAD)

    zs2d, denom2 = _pass1(ep2d, cars1_flat, ent_flat, meta_flat,
                          eutab, evtab, consts)
    carsp, entp = _pass2(ep2d, zs2d, cars1_flat, denom2)
    cars_new, ent_new = _combine(carsp, entp)

    return (cars_new[:N, None], emb_out[:N], ent_new[:N, None])


# speedup vs baseline: 192.9708x; 1.3389x over previous
import jax
import jax.numpy as jnp
from jax import lax
from jax.experimental import pallas as pl
from jax.experimental.pallas import tpu as pltpu
from jax.experimental.pallas import tpu_sc as plsc

N = 100000
E = 1600000
EMB = 32
VOCAB = 1000

N_PAD = 102400
NROWS = N_PAD // 128
NCHN = N_PAD // 512
SL = N_PAD // 16

EC = 2560
NCHE = E // EC
T2 = 10

_f32 = jnp.float32
_i32 = jnp.int32

_SC_PARAMS = pltpu.CompilerParams(needs_layout_passes=False,
                                  use_tc_tiling_on_sc=False)


def _tables_body(embed_ref, w2p_ref, b2p_ref, w2e_ref, b2e_ref, w3_ref,
                 b3_ref, eu_ref, ev_ref, q_ref):
    emb = embed_ref[...]
    w3m = w3_ref[...][:32]
    u = jnp.dot(w2e_ref[...][:32], w3m, preferred_element_type=_f32)
    v = jnp.dot(w2e_ref[...][32:], w3m, preferred_element_type=_f32)
    c8 = jnp.dot(b2e_ref[...].reshape(1, 32), w3m,
                 preferred_element_type=_f32) + b3_ref[...].reshape(1, 8)
    eu_ref[...] = jnp.dot(emb, u, preferred_element_type=_f32) + c8
    ev_ref[...] = jnp.dot(emb, v, preferred_element_type=_f32)
    q_ref[...] = jnp.dot(emb, w2p_ref[...][:32],
                         preferred_element_type=_f32) + b2p_ref[...]


def _make_tables(embed, W2p, b2p, W2e, b2e, W3, b3):
    return pl.pallas_call(
        _tables_body,
        out_shape=(
            jax.ShapeDtypeStruct((VOCAB, 8), _f32),
            jax.ShapeDtypeStruct((VOCAB, 8), _f32),
            jax.ShapeDtypeStruct((VOCAB, 1), _f32),
        ),
    )(embed, W2p, b2p, W2e, b2e, W3, b3)


def _mesh():
    return plsc.VectorSubcoreMesh(core_axis_name="c", subcore_axis_name="s")


def _zero_shared(zeroblk, sh, base):
    def zb(i, _):
        zeroblk[pl.ds(i * 16, 16)] = jnp.zeros((16,), _f32)
        return 0
    lax.fori_loop(0, 128, zb, 0)
    for off in (0, 2048, 4096):
        pltpu.sync_copy(zeroblk, sh.at[pl.ds(base + off, 2048)])
    pltpu.sync_copy(zeroblk.at[pl.ds(0, 256)], sh.at[pl.ds(base + 6144, 256)])


def _node_body(nf_hbm, cars_hbm, free_hbm, q_hbm, embed_hbm, consts_hbm,
               emb_out, cars1_out, meta_out,
               qtab, cbuf, nfbuf, carsbuf, freebuf, c1buf, mbuf, embrows,
               sem):
    cid = lax.axis_index("c")
    sid = lax.axis_index("s")
    wid = sid * 2 + cid
    pltpu.sync_copy(q_hbm, qtab)
    pltpu.sync_copy(consts_hbm, cbuf)
    w2p_last = cbuf[5, pl.ds(0, 16)][0]

    def chunk(t, _):
        c = wid + 32 * t

        @pl.when(c < NCHN)
        def _():
            r4 = c * 4
            pltpu.sync_copy(nf_hbm.at[pl.ds(r4, 4)], nfbuf)
            pltpu.sync_copy(cars_hbm.at[pl.ds(r4, 4)], carsbuf)
            pltpu.sync_copy(free_hbm.at[pl.ds(r4, 4)], freebuf)
            descs = [
                pltpu.async_copy(embed_hbm.at[nfbuf.at[j]],
                                 embrows.at[pl.ds(j * 128, 128)], sem)
                for j in range(4)
            ]
            for dsc in descs:
                dsc.wait()
            pltpu.sync_copy(embrows, emb_out.at[pl.ds(c * 512, 512)])

            def grp(g, _):
                j = g // 8
                k = (g % 8) * 16
                nfv = nfbuf[j, pl.ds(k, 16)]
                cv = carsbuf[j, pl.ds(k, 16)]
                fv = freebuf[j, pl.ds(k, 16)]
                qg = plsc.load_gather(qtab, [nfv, jnp.zeros((16,), _i32)])
                parked = qg + cv * w2p_last
                c1 = jnp.maximum(jnp.maximum(parked, 0.0) + cv, 0.0)
                meta = nfv + jnp.where(fv > 0.5, 1024, 0).astype(_i32)
                c1buf[j, pl.ds(k, 16)] = c1
                mbuf[j, pl.ds(k, 16)] = meta
                return 0

            lax.fori_loop(0, 32, grp, 0)
            pltpu.sync_copy(c1buf, cars1_out.at[pl.ds(r4, 4)])
            pltpu.sync_copy(mbuf, meta_out.at[pl.ds(r4, 4)])

        return 0

    lax.fori_loop(0, (NCHN + 31) // 32, chunk, 0)


def _node_stage(nf2d, cars2d, free2d, qtab, embed, consts):
    fn = pl.kernel(
        _node_body,
        out_type=(
            jax.ShapeDtypeStruct((N_PAD, EMB), _f32),
            jax.ShapeDtypeStruct((NROWS, 128), _f32),
            jax.ShapeDtypeStruct((NROWS, 128), _i32),
        ),
        mesh=_mesh(),
        compiler_params=_SC_PARAMS,
        scratch_types=(
            pltpu.VMEM((VOCAB, 1), _f32),
            pltpu.VMEM((8, 16), _f32),
            pltpu.VMEM((4, 128), _i32),
            pltpu.VMEM((4, 128), _f32),
            pltpu.VMEM((4, 128), _f32),
            pltpu.VMEM((4, 128), _f32),
            pltpu.VMEM((4, 128), _i32),
            pltpu.VMEM((512, EMB), _f32),
            pltpu.SemaphoreType.DMA,
        ),
    )
    return fn(nf2d, cars2d, free2d, qtab, embed, consts)


def _pass1_body(ep_hbm, c1_hbm, ent_hbm, meta_hbm, eu_hbm, ev_hbm, consts_hbm,
                zs_out, denom_out,
                eutab, evtab, cbuf, zeroblk,
                sbuf0, dbuf0, gc1s0, gents0, gmets0, gc1d0, gmetd0, zp0, zb0,
                sbuf1, dbuf1, gc1s1, gents1, gmets1, gc1d1, gmetd1, zp1, zb1,
                gsem0, gsem1,
                sh_c1, sh_ent, sh_meta, sh_denom):
    cid = lax.axis_index("c")
    sid = lax.axis_index("s")
    wid = sid * 2 + cid
    base = sid * SL

    pltpu.sync_copy(eu_hbm, eutab)
    pltpu.sync_copy(ev_hbm, evtab)
    pltpu.sync_copy(consts_hbm, cbuf)
    pltpu.sync_copy(c1_hbm.at[pl.ds(base, SL)], sh_c1.at[pl.ds(base, SL)])
    pltpu.sync_copy(ent_hbm.at[pl.ds(base, SL)], sh_ent.at[pl.ds(base, SL)])
    pltpu.sync_copy(meta_hbm.at[pl.ds(base, SL)], sh_meta.at[pl.ds(base, SL)])
    _zero_shared(zeroblk, sh_denom, base)
    plsc.subcore_barrier()

    r0 = cbuf[0, pl.ds(0, 16)]
    r1 = cbuf[1, pl.ds(0, 16)]
    r2 = cbuf[2, pl.ds(0, 16)]
    r3 = cbuf[3, pl.ds(0, 16)]
    r4c = cbuf[4, pl.ds(0, 16)]
    w31 = [r0[c] for c in range(8)]
    w32 = [r1[c] for c in range(8)]
    w33 = [r2[c] for c in range(8)]
    w4 = [r3[c] for c in range(8)]
    b4 = r4c[0]

    bufs = (
        (sbuf0, dbuf0, gc1s0, gents0, gmets0, gc1d0, gmetd0, zp0, zb0, gsem0),
        (sbuf1, dbuf1, gc1s1, gents1, gmets1, gc1d1, gmetd1, zp1, zb1, gsem1),
    )

    def gdescs(B):
        sb, db, c1s, ents, mets, c1d, metd, zp, zb, sem = B
        return [
            pltpu.make_async_copy(sh_c1.at[sb], c1s, sem),
            pltpu.make_async_copy(sh_ent.at[sb], ents, sem),
            pltpu.make_async_copy(sh_meta.at[sb], mets, sem),
            pltpu.make_async_copy(sh_c1.at[db], c1d, sem),
            pltpu.make_async_copy(sh_meta.at[db], metd, sem),
        ]

    def issue(c, B):
        sb, db = B[0], B[1]
        pltpu.sync_copy(ep_hbm.at[0, pl.ds(c * EC, EC)], sb)
        pltpu.sync_copy(ep_hbm.at[1, pl.ds(c * EC, EC)], db)
        for d in gdescs(B):
            d.start()

    def consume(c, B):
        sb, db, c1s, ents, mets, c1d, metd, zp, zb, sem = B
        for d in gdescs(B):
            d.wait()

        def grp(g, _):
            s16 = pl.ds(g * 16, 16)
            c1sv = c1s[s16]
            entsv = ents[s16]
            metsv = mets[s16]
            c1dv = c1d[s16]
            metdv = metd[s16]
            nfs = jnp.bitwise_and(metsv, 1023)
            frees = lax.shift_right_logical(metsv, 10)
            nfd = jnp.bitwise_and(metdv, 1023)
            h3 = jnp.full((16,), 0.0, _f32) + b4
            for cc in range(8):
                ccv = jnp.full((16,), cc, _i32)
                eu = plsc.load_gather(eutab, [nfs, ccv])
                ev = plsc.load_gather(evtab, [nfd, ccv])
                a = (eu + ev + c1sv * w31[cc] + c1dv * w32[cc]
                     + entsv * w33[cc])
                h3 = h3 + jnp.maximum(a, 0.0) * w4[cc]
            lf = jnp.maximum(h3, 0.0)
            slb = nfs == nfd
            enb = slb != (frees == 1)
            lf = jnp.where(enb, lf, 0.0)
            z = jnp.exp(lf)
            zp[s16] = z
            zb[s16] = jnp.where(slb, -z, z)
            return 0

        lax.fori_loop(0, EC // 16, grp, 0)
        sc = pltpu.async_copy(zp, sh_denom.at[sb], sem, add=True)
        pltpu.sync_copy(zb, zs_out.at[pl.ds(c * EC, EC)])
        sc.wait()

    issue(wid, bufs[0])

    def outer(t2, _):
        for b in (0, 1):
            c = wid + 32 * (2 * t2 + b)
            cn = c + 32

            @pl.when(cn < NCHE)
            def _():
                issue(cn, bufs[1 - b])

            @pl.when(c < NCHE)
            def _():
                consume(c, bufs[b])

        return 0

    lax.fori_loop(0, T2, outer, 0)
    plsc.subcore_barrier()
    pltpu.sync_copy(sh_denom.at[pl.ds(base, SL)],
                    denom_out.at[cid, pl.ds(base, SL)])


def _pass1(ep, cars1_flat, ent_flat, meta_flat, eutab, evtab, consts):
    ebufs = []
    for _ in range(2):
        ebufs += [
            pltpu.VMEM((EC,), _i32),
            pltpu.VMEM((EC,), _i32),
            pltpu.VMEM((EC,), _f32),
            pltpu.VMEM((EC,), _f32),
            pltpu.VMEM((EC,), _i32),
            pltpu.VMEM((EC,), _f32),
            pltpu.VMEM((EC,), _i32),
            pltpu.VMEM((EC,), _f32),
            pltpu.VMEM((EC,), _f32),
        ]
    fn = pl.kernel(
        _pass1_body,
        out_type=(
            jax.ShapeDtypeStruct((E,), _f32),
            jax.ShapeDtypeStruct((2, N_PAD), _f32),
        ),
        mesh=_mesh(),
        compiler_params=_SC_PARAMS,
        scratch_types=(
            pltpu.VMEM((VOCAB, 8), _f32),
            pltpu.VMEM((VOCAB, 8), _f32),
            pltpu.VMEM((8, 16), _f32),
            pltpu.VMEM((2048,), _f32),
            *ebufs,
            pltpu.SemaphoreType.DMA,
            pltpu.SemaphoreType.DMA,
            pltpu.VMEM_SHARED((N_PAD,), _f32),
            pltpu.VMEM_SHARED((N_PAD,), _f32),
            pltpu.VMEM_SHARED((N_PAD,), _i32),
            pltpu.VMEM_SHARED((N_PAD,), _f32),
        ),
    )
    return fn(ep, cars1_flat, ent_flat, meta_flat, eutab, evtab, consts)


def _pass2_body(ep_hbm, zs_hbm, c1_hbm, denom_hbm,
                cars_out, ent_out,
                d0buf, d1buf, c1chunk, wbuf, zeroblk,
                sbuf0, dbuf0, zsb0, wsg0, msgb0, outb0,
                sbuf1, dbuf1, zsb1, wsg1, msgb1, outb1,
                gsem0, gsem1,
                sh_w, sh_cars, sh_ent):
    cid = lax.axis_index("c")
    sid = lax.axis_index("s")
    wid = sid * 2 + cid
    base = sid * SL

    for off, sz in ((0, 2048), (2048, 2048), (4096, 2048), (6144, 256)):
        pltpu.sync_copy(denom_hbm.at[0, pl.ds(base + off, sz)],
                        d0buf.at[pl.ds(0, sz)])
        pltpu.sync_copy(denom_hbm.at[1, pl.ds(base + off, sz)],
                        d1buf.at[pl.ds(0, sz)])
        pltpu.sync_copy(c1_hbm.at[pl.ds(base + off, sz)],
                        c1chunk.at[pl.ds(0, sz)])

        def wgrp(i, _):
            s16 = pl.ds(i * 16, 16)
            wbuf[s16] = c1chunk[s16] / (d0buf[s16] + d1buf[s16])
            return 0

        lax.fori_loop(0, sz // 16, wgrp, 0)
        pltpu.sync_copy(wbuf.at[pl.ds(0, sz)],
                        sh_w.at[pl.ds(base + off, sz)])

    _zero_shared(zeroblk, sh_cars, base)
    _zero_shared(zeroblk, sh_ent, base)
    plsc.subcore_barrier()

    bufs = (
        (sbuf0, dbuf0, zsb0, wsg0, msgb0, outb0, gsem0),
        (sbuf1, dbuf1, zsb1, wsg1, msgb1, outb1, gsem1),
    )

    def issue(c, B):
        sb, db, zsb, wsg, msgb, outb, sem = B
        pltpu.sync_copy(ep_hbm.at[0, pl.ds(c * EC, EC)], sb)
        pltpu.sync_copy(ep_hbm.at[1, pl.ds(c * EC, EC)], db)
        pltpu.sync_copy(zs_hbm.at[pl.ds(c * EC, EC)], zsb)
        pltpu.make_async_copy(sh_w.at[sb], wsg, sem).start()

    def consume(c, B):
        sb, db, zsb, wsg, msgb, outb, sem = B
        pltpu.make_async_copy(sh_w.at[sb], wsg, sem).wait()

        def grp(g, _):
            s16 = pl.ds(g * 16, 16)
            zs16 = zsb[s16]
            ws = wsg[s16]
            slb = zs16 < 0.0
            msg = jnp.abs(zs16) * ws
            msgb[s16] = msg
            outb[s16] = jnp.where(slb, 0.0, msg)
            return 0

        lax.fori_loop(0, EC // 16, grp, 0)
        sc1 = pltpu.async_copy(msgb, sh_cars.at[db], sem, add=True)
        sc2 = pltpu.async_copy(outb, sh_ent.at[db], sem, add=True)
        sc1.wait()
        sc2.wait()

    issue(wid, bufs[0])

    def outer(t2, _):
        for b in (0, 1):
            c = wid + 32 * (2 * t2 + b)
            cn = c + 32

            @pl.when(cn < NCHE)
            def _():
                issue(cn, bufs[1 - b])

            @pl.when(c < NCHE)
            def _():
                consume(c, bufs[b])

        return 0

    lax.fori_loop(0, T2, outer, 0)
    plsc.subcore_barrier()
    pltpu.sync_copy(sh_cars.at[pl.ds(base, SL)],
                    cars_out.at[cid, pl.ds(base, SL)])
    pltpu.sync_copy(sh_ent.at[pl.ds(base, SL)],
                    ent_out.at[cid, pl.ds(base, SL)])


def _pass2(ep, zs, cars1_flat, denom2):
    ebufs = []
    for _ in range(2):
        ebufs += [
            pltpu.VMEM((EC,), _i32),
            pltpu.VMEM((EC,), _i32),
            pltpu.VMEM((EC,), _f32),
            pltpu.VMEM((EC,), _f32),
            pltpu.VMEM((EC,), _f32),
            pltpu.VMEM((EC,), _f32),
        ]
    fn = pl.kernel(
        _pass2_body,
        out_type=(
            jax.ShapeDtypeStruct((2, N_PAD), _f32),
            jax.ShapeDtypeStruct((2, N_PAD), _f32),
        ),
        mesh=_mesh(),
        compiler_params=_SC_PARAMS,
        scratch_types=(
            pltpu.VMEM((2048,), _f32),
            pltpu.VMEM((2048,), _f32),
            pltpu.VMEM((2048,), _f32),
            pltpu.VMEM((2048,), _f32),
            pltpu.VMEM((2048,), _f32),
            *ebufs,
            pltpu.SemaphoreType.DMA,
            pltpu.SemaphoreType.DMA,
            pltpu.VMEM_SHARED((N_PAD,), _f32),
            pltpu.VMEM_SHARED((N_PAD,), _f32),
            pltpu.VMEM_SHARED((N_PAD,), _f32),
        ),
    )
    return fn(ep, zs, cars1_flat, denom2)


def _combine_body(a_ref, b_ref, ca_ref, cb_ref):
    ca_ref[...] = jnp.sum(a_ref[...], axis=0)
    cb_ref[...] = jnp.sum(b_ref[...], axis=0)


def _combine(carsp, entp):
    blk = 2048
    return pl.pallas_call(
        _combine_body,
        grid=(N_PAD // blk,),
        in_specs=[
            pl.BlockSpec((2, blk), lambda i: (0, i)),
            pl.BlockSpec((2, blk), lambda i: (0, i)),
        ],
        out_specs=[
            pl.BlockSpec((blk,), lambda i: (i,)),
            pl.BlockSpec((blk,), lambda i: (i,)),
        ],
        out_shape=(
            jax.ShapeDtypeStruct((N_PAD,), _f32),
            jax.ShapeDtypeStruct((N_PAD,), _f32),
        ),
    )(carsp, entp)


def kernel(edge_index, nfeatures, cars, free, entered, embed,
           W2p, b2p, W2e, b2e, W3, b3, W4, b4):
    eutab, evtab, q2d = _make_tables(embed, W2p, b2p, W2e, b2e, W3, b3)

    consts = jnp.zeros((8, 16), _f32)
    consts = consts.at[0, :8].set(W3[32])
    consts = consts.at[1, :8].set(W3[33])
    consts = consts.at[2, :8].set(W3[34])
    consts = consts.at[3, :8].set(W4[:, 0])
    consts = consts.at[4, 0].set(b4[0])
    consts = consts.at[5, 0].set(W2p[32, 0])

    nf2d = jnp.pad(nfeatures, (0, N_PAD - N)).reshape(NROWS, 128)
    cars2d = jnp.pad(cars[:, 0], (0, N_PAD - N)).reshape(NROWS, 128)
    free2d = jnp.pad(free[:, 0], (0, N_PAD - N)).reshape(NROWS, 128)
    ent_flat = jnp.pad(entered[:, 0], (0, N_PAD - N))

    emb_out, cars1_2d, meta2d = _node_stage(nf2d, cars2d, free2d, q2d,
                                            embed, consts)
    cars1_flat = cars1_2d.reshape(N_PAD)
    meta_flat = meta2d.reshape(N_PAD)

    zs, denom2 = _pass1(edge_index, cars1_flat, ent_flat, meta_flat,
                        eutab, evtab, consts)
    carsp, entp = _pass2(edge_index, zs, cars1_flat, denom2)
    cars_new, ent_new = _combine(carsp, entp)

    return (cars_new[:N, None], emb_out[:N], ent_new[:N, None])
